# Initial kernel scaffold; baseline (speedup 1.0000x reference)
import functools

import jax
import jax.numpy as jnp
from jax import lax
from jax.experimental import pallas as pl
from jax.experimental.pallas import tpu as pltpu
from jax.experimental.pallas import tpu_sc as plsc

_NUM_CORES = 2
_NUM_SUBCORES = 16
_NUM_WORKERS = _NUM_CORES * _NUM_SUBCORES

_VOCAB = 1000000
_EMBED_DIM = 64
_BATCH = 16384
_HIST = 50
_TOTAL = _BATCH * _HIST

_CHUNK = 128
_PER_WORKER = _TOTAL // _NUM_WORKERS
_NUM_CHUNKS = _PER_WORKER // _CHUNK


def _make_gather():
    mesh = plsc.VectorSubcoreMesh(
        core_axis_name="c",
        subcore_axis_name="s",
        num_cores=_NUM_CORES,
        num_subcores=_NUM_SUBCORES,
    )

    @functools.partial(
        pl.kernel,
        out_type=jax.ShapeDtypeStruct((_TOTAL, _EMBED_DIM), jnp.float32),
        mesh=mesh,
        scratch_types=[
            pltpu.VMEM((_NUM_CHUNKS, _CHUNK), jnp.int32),
            pltpu.VMEM((_CHUNK, _EMBED_DIM), jnp.float32),
            pltpu.SemaphoreType.DMA,
        ],
    )
    def gather_kernel(idx_hbm, table_hbm, out_hbm, idx_v, rows_v, sem):
        wid = lax.axis_index("s") * _NUM_CORES + lax.axis_index("c")
        base = wid * _PER_WORKER
        pltpu.sync_copy(idx_hbm.at[wid], idx_v)

        def body(j, carry):
            pltpu.async_copy(table_hbm.at[idx_v.at[j]], rows_v, sem).wait()
            pltpu.sync_copy(rows_v, out_hbm.at[pl.ds(base + j * _CHUNK, _CHUNK)])
            return carry

        lax.fori_loop(0, _NUM_CHUNKS, body, 0, unroll=False)

    return gather_kernel


_gather = _make_gather()


def kernel(words, emb_weight):
    idx = words.reshape(_NUM_WORKERS, _NUM_CHUNKS, _CHUNK).astype(jnp.int32)
    out = _gather(idx, emb_weight)
    return out.reshape(_BATCH, _HIST, _EMBED_DIM)


# speedup vs baseline: 1.6836x; 1.6836x over previous
import functools

import jax
import jax.numpy as jnp
from jax import lax
from jax.experimental import pallas as pl
from jax.experimental.pallas import tpu as pltpu
from jax.experimental.pallas import tpu_sc as plsc

_NUM_CORES = 2
_NUM_SUBCORES = 16
_NUM_WORKERS = _NUM_CORES * _NUM_SUBCORES

_VOCAB = 1000000
_EMBED_DIM = 64
_BATCH = 16384
_HIST = 50
_TOTAL = _BATCH * _HIST

_CHUNK = 128
_PER_WORKER = _TOTAL // _NUM_WORKERS
_NUM_CHUNKS = _PER_WORKER // _CHUNK


def _make_gather():
    mesh = plsc.VectorSubcoreMesh(
        core_axis_name="c",
        subcore_axis_name="s",
        num_cores=_NUM_CORES,
        num_subcores=_NUM_SUBCORES,
    )

    @functools.partial(
        pl.kernel,
        out_type=jax.ShapeDtypeStruct((_TOTAL, _EMBED_DIM), jnp.float32),
        mesh=mesh,
        scratch_types=[
            pltpu.VMEM((_NUM_CHUNKS, _CHUNK), jnp.int32),
            pltpu.VMEM((_CHUNK, _EMBED_DIM), jnp.float32),
            pltpu.SemaphoreType.DMA,
        ],
        compiler_params=pltpu.CompilerParams(use_tc_tiling_on_sc=False),
    )
    def gather_kernel(idx_hbm, table_hbm, out_hbm, idx_v, rows_v, sem):
        wid = lax.axis_index("s") * _NUM_CORES + lax.axis_index("c")
        base = wid * _PER_WORKER
        pltpu.sync_copy(idx_hbm.at[wid], idx_v)

        def body(j, carry):
            pltpu.async_copy(table_hbm.at[idx_v.at[j]], rows_v, sem).wait()
            pltpu.sync_copy(rows_v, out_hbm.at[pl.ds(base + j * _CHUNK, _CHUNK)])
            return carry

        lax.fori_loop(0, _NUM_CHUNKS, body, 0, unroll=False)

    return gather_kernel


_gather = _make_gather()


def kernel(words, emb_weight):
    idx = words.reshape(_NUM_WORKERS, _NUM_CHUNKS, _CHUNK).astype(jnp.int32)
    out = _gather(idx, emb_weight)
    return out.reshape(_BATCH, _HIST, _EMBED_DIM)


# speedup vs baseline: 1.8698x; 1.1106x over previous
import functools

import jax
import jax.numpy as jnp
from jax import lax
from jax.experimental import pallas as pl
from jax.experimental.pallas import tpu as pltpu
from jax.experimental.pallas import tpu_sc as plsc

_NUM_CORES = 2
_NUM_SUBCORES = 16
_NUM_WORKERS = _NUM_CORES * _NUM_SUBCORES

_EMBED_DIM = 64
_BATCH = 16384
_HIST = 50
_TOTAL = _BATCH * _HIST

_CHUNK = 128
_K = 4
_PER_WORKER = _TOTAL // _NUM_WORKERS
_NUM_CHUNKS = _PER_WORKER // _CHUNK
_NUM_BLOCKS = _NUM_CHUNKS // _K


def _make_gather():
    mesh = plsc.VectorSubcoreMesh(
        core_axis_name="c",
        subcore_axis_name="s",
        num_cores=_NUM_CORES,
        num_subcores=_NUM_SUBCORES,
    )

    @functools.partial(
        pl.kernel,
        out_type=jax.ShapeDtypeStruct((_TOTAL, _EMBED_DIM), jnp.float32),
        mesh=mesh,
        scratch_types=[
            pltpu.VMEM((_NUM_CHUNKS, _CHUNK), jnp.int32),
            pltpu.VMEM((2, _K, _CHUNK, _EMBED_DIM), jnp.float32),
            pltpu.SemaphoreType.DMA,
            pltpu.SemaphoreType.DMA,
        ],
        compiler_params=pltpu.CompilerParams(use_tc_tiling_on_sc=False),
    )
    def gather_kernel(idx_hbm, table_hbm, out_hbm, idx_v, rows_v, gsem, ssem):
        wid = lax.axis_index("s") * _NUM_CORES + lax.axis_index("c")
        base = wid * _PER_WORKER
        pltpu.sync_copy(idx_hbm.at[wid], idx_v)

        def fire_gathers(t, grp):
            for b in range(_K):
                pltpu.async_copy(
                    table_hbm.at[idx_v.at[t * _K + b]], rows_v.at[grp, b], gsem
                )

        def drain(sem, is_store):
            if is_store:
                pltpu.make_async_copy(
                    rows_v.at[0, 0], out_hbm.at[pl.ds(base, _CHUNK)], sem
                ).wait()
            else:
                pltpu.make_async_copy(
                    table_hbm.at[idx_v.at[0]], rows_v.at[0, 0], sem
                ).wait()

        fire_gathers(0, 0)

        def body(t, carry):
            grp = lax.rem(t, 2)
            for _ in range(_K):
                drain(gsem, is_store=False)
            @pl.when(t >= 1)
            def _():
                for _ in range(_K):
                    drain(ssem, is_store=True)

            @pl.when(t + 1 < _NUM_BLOCKS)
            def _():
                fire_gathers(t + 1, 1 - grp)

            for b in range(_K):
                pltpu.async_copy(
                    rows_v.at[grp, b],
                    out_hbm.at[pl.ds(base + (t * _K + b) * _CHUNK, _CHUNK)],
                    ssem,
                )
            return carry

        lax.fori_loop(0, _NUM_BLOCKS, body, 0, unroll=False)
        for _ in range(_K):
            drain(ssem, is_store=True)

    return gather_kernel


_gather = _make_gather()


def kernel(words, emb_weight):
    idx = words.reshape(_NUM_WORKERS, _NUM_CHUNKS, _CHUNK).astype(jnp.int32)
    out = _gather(idx, emb_weight)
    return out.reshape(_BATCH, _HIST, _EMBED_DIM)
